# Initial kernel scaffold; baseline (speedup 1.0000x reference)
import jax
import jax.numpy as jnp
from jax.experimental import pallas as pl


def kernel(emb, or_src, or_dst, epoch):
    raise NotImplementedError("write your pallas kernel here")


# speedup vs baseline: 310.0087x; 310.0087x over previous
import functools

import jax
import jax.numpy as jnp
from jax import lax
from jax.experimental import pallas as pl
from jax.experimental.pallas import tpu as pltpu
from jax.experimental.pallas import tpu_sc as plsc

NV = 50000
N_LIT = 2 * NV
N_CLAUSES = 200000
E = 1600000
INV_T = float(2.0 ** 1.2)

NCHUNK = E // 128
NW = 32
BASE_CH = NCHUNK // NW
EXTRA = NCHUNK - BASE_CH * NW
GE = 4
FULL_GROUPS = BASE_CH // GE

ACC = 200192
ACC_SL = ACC // 16
ZHALF = ACC_SL // 2
EMB_SL = 3120
EMB_CP = 3200


def _sc_body(emb_h, src_h, dst_h, out_h,
             lit_sp, num_sp, den_sp,
             embw, negw, zbuf, src_b, dst_b, srct, dstt,
             v_row, a_row, b_row):
    c = lax.axis_index("c")
    s = lax.axis_index("s")
    wid = c * 16 + s

    def z16(i, _):
        zbuf[pl.ds(i * 16, 16)] = jnp.zeros((16,), jnp.float32)
        return _
    lax.fori_loop(0, ZHALF // 16, z16, None)
    zb = s * ACC_SL
    pltpu.sync_copy(zbuf, num_sp.at[pl.ds(zb, ZHALF)])
    pltpu.sync_copy(zbuf, num_sp.at[pl.ds(zb + ZHALF, ZHALF)])
    pltpu.sync_copy(zbuf, den_sp.at[pl.ds(zb, ZHALF)])
    pltpu.sync_copy(zbuf, den_sp.at[pl.ds(zb + ZHALF, ZHALF)])

    off = s * EMB_SL
    pltpu.sync_copy(emb_h.at[pl.ds(off, EMB_CP)], embw)

    def neg(i, _):
        negw[pl.ds(i * 16, 16)] = 1.0 - embw[pl.ds(i * 16, 16)]
        return _
    lax.fori_loop(0, EMB_CP // 16, neg, None)
    pltpu.sync_copy(embw, lit_sp.at[pl.ds(off, EMB_CP)])
    pltpu.sync_copy(negw, lit_sp.at[pl.ds(NV + off, EMB_CP)])
    plsc.subcore_barrier()

    cb = wid * BASE_CH + jnp.minimum(wid, EXTRA)
    nch = jnp.where(wid < EXTRA, BASE_CH + 1, BASE_CH)

    def process_chunk(src_row, dst_row):
        pltpu.sync_copy(lit_sp.at[src_row], v_row)
        for i in range(8):
            sl = pl.ds(i * 16, 16)
            v = v_row[sl]
            e = jnp.exp(v * INV_T)
            a_row[sl] = v * e
            b_row[sl] = e
        pltpu.sync_copy(a_row, num_sp.at[dst_row], add=True)
        pltpu.sync_copy(b_row, den_sp.at[dst_row], add=True)

    def grp(g, _):
        rb = cb + g * GE
        pltpu.sync_copy(src_h.at[pl.ds(rb, GE)], src_b)
        pltpu.sync_copy(dst_h.at[pl.ds(rb, GE)], dst_b)
        for j in range(GE):
            process_chunk(src_b.at[j], dst_b.at[j])
        return _
    lax.fori_loop(0, FULL_GROUPS, grp, None)

    def tail(j, _):
        rb = cb + FULL_GROUPS * GE + j
        pltpu.sync_copy(src_h.at[pl.ds(rb, 1)], srct)
        pltpu.sync_copy(dst_h.at[pl.ds(rb, 1)], dstt)
        process_chunk(srct.at[0], dstt.at[0])
        return _
    lax.fori_loop(0, nch - FULL_GROUPS * GE, tail, None)
    plsc.subcore_barrier()

    wb = s * ACC_SL
    pltpu.sync_copy(num_sp.at[pl.ds(wb, ACC_SL)], out_h.at[c, 0, pl.ds(wb, ACC_SL)])
    pltpu.sync_copy(den_sp.at[pl.ds(wb, ACC_SL)], out_h.at[c, 1, pl.ds(wb, ACC_SL)])


_sc_edge_pass = functools.partial(
    pl.kernel,
    out_type=jax.ShapeDtypeStruct((2, 2, ACC), jnp.float32),
    mesh=plsc.VectorSubcoreMesh(core_axis_name="c", subcore_axis_name="s"),
    compiler_params=pltpu.CompilerParams(use_tc_tiling_on_sc=False),
    scratch_types=[
        pltpu.VMEM_SHARED((N_LIT,), jnp.float32),
        pltpu.VMEM_SHARED((ACC,), jnp.float32),
        pltpu.VMEM_SHARED((ACC,), jnp.float32),
        pltpu.VMEM((EMB_CP,), jnp.float32),
        pltpu.VMEM((EMB_CP,), jnp.float32),
        pltpu.VMEM((ZHALF,), jnp.float32),
        pltpu.VMEM((GE, 128), jnp.int32),
        pltpu.VMEM((GE, 128), jnp.int32),
        pltpu.VMEM((1, 128), jnp.int32),
        pltpu.VMEM((1, 128), jnp.int32),
        pltpu.VMEM((128,), jnp.float32),
        pltpu.VMEM((128,), jnp.float32),
        pltpu.VMEM((128,), jnp.float32),
    ],
)(_sc_body)


_FIN_ROWS = ACC // 128


def _fin_body(p_ref, o_ref):
    num = p_ref[0, 0] + p_ref[1, 0]
    den = p_ref[0, 1] + p_ref[1, 1]
    row = lax.broadcasted_iota(jnp.int32, (_FIN_ROWS, 128), 0)
    col = lax.broadcasted_iota(jnp.int32, (_FIN_ROWS, 128), 1)
    idx = row * 128 + col
    valid = idx < N_CLAUSES
    cval = jnp.where(valid & (den > 0), num / jnp.maximum(den, 1e-30), 0.0)
    w2 = jnp.where(valid, jnp.exp(-cval * INV_T), 0.0)
    o_ref[...] = (jnp.sum(cval * w2) / jnp.sum(w2)).reshape(1, 1)


_finalize = pl.pallas_call(
    _fin_body,
    out_shape=jax.ShapeDtypeStruct((1, 1), jnp.float32),
)


def kernel(emb, or_src, or_dst, epoch):
    del epoch
    src2 = or_src.reshape(NCHUNK, 128)
    dst2 = or_dst.reshape(NCHUNK, 128)
    part = _sc_edge_pass(emb, src2, dst2)
    res = _finalize(part.reshape(2, 2, _FIN_ROWS, 128))
    return res[0, 0]


# speedup vs baseline: 525.7653x; 1.6960x over previous
import functools

import jax
import jax.numpy as jnp
from jax import lax
from jax.experimental import pallas as pl
from jax.experimental.pallas import tpu as pltpu
from jax.experimental.pallas import tpu_sc as plsc

NV = 50000
N_LIT = 2 * NV
N_CLAUSES = 200000
E = 1600000
INV_T = float(2.0 ** 1.2)

NCHUNK = E // 128
NW = 32
BASE_CH = NCHUNK // NW
EXTRA = NCHUNK - BASE_CH * NW
GE = 8
GEL = GE * 128
FULL_GROUPS = BASE_CH // GE

ACC = 200192
ACC_SL = ACC // 16
ZHALF = ACC_SL // 2
EMB_SL = 3120
EMB_CP = 3200


def _sc_body(emb_h, src_h, dst_h, out_h,
             lit_sp, num_sp, den_sp,
             embw, negw, zbuf, src_b, dst_b, srct, dstt,
             v_b, a_b, b_b, v_row, a_row, b_row):
    c = lax.axis_index("c")
    s = lax.axis_index("s")
    wid = c * 16 + s

    def z16(i, _):
        zbuf[pl.ds(i * 16, 16)] = jnp.zeros((16,), jnp.float32)
        return _
    lax.fori_loop(0, ZHALF // 16, z16, None)
    zb = s * ACC_SL
    pltpu.sync_copy(zbuf, num_sp.at[pl.ds(zb, ZHALF)])
    pltpu.sync_copy(zbuf, num_sp.at[pl.ds(zb + ZHALF, ZHALF)])
    pltpu.sync_copy(zbuf, den_sp.at[pl.ds(zb, ZHALF)])
    pltpu.sync_copy(zbuf, den_sp.at[pl.ds(zb + ZHALF, ZHALF)])

    off = s * EMB_SL
    pltpu.sync_copy(emb_h.at[pl.ds(off, EMB_CP)], embw)

    def neg(i, _):
        negw[pl.ds(i * 16, 16)] = 1.0 - embw[pl.ds(i * 16, 16)]
        return _
    lax.fori_loop(0, EMB_CP // 16, neg, None)
    pltpu.sync_copy(embw, lit_sp.at[pl.ds(off, EMB_CP)])
    pltpu.sync_copy(negw, lit_sp.at[pl.ds(NV + off, EMB_CP)])
    plsc.subcore_barrier()

    cb = wid * BASE_CH + jnp.minimum(wid, EXTRA)
    nch = jnp.where(wid < EXTRA, BASE_CH + 1, BASE_CH)
    eb = cb * 128

    def grp(g, _):
        rb = eb + g * GEL
        pltpu.sync_copy(src_h.at[pl.ds(rb, GEL)], src_b)
        pltpu.sync_copy(dst_h.at[pl.ds(rb, GEL)], dst_b)
        pltpu.sync_copy(lit_sp.at[src_b], v_b)
        for i in range(GEL // 16):
            sl = pl.ds(i * 16, 16)
            v = v_b[sl]
            e = jnp.exp(v * INV_T)
            a_b[sl] = v * e
            b_b[sl] = e
        pltpu.sync_copy(a_b, num_sp.at[dst_b], add=True)
        pltpu.sync_copy(b_b, den_sp.at[dst_b], add=True)
        return _
    lax.fori_loop(0, FULL_GROUPS, grp, None)

    def tail(j, _):
        rb = eb + FULL_GROUPS * GEL + j * 128
        pltpu.sync_copy(src_h.at[pl.ds(rb, 128)], srct)
        pltpu.sync_copy(dst_h.at[pl.ds(rb, 128)], dstt)
        pltpu.sync_copy(lit_sp.at[srct], v_row)
        for i in range(8):
            sl = pl.ds(i * 16, 16)
            v = v_row[sl]
            e = jnp.exp(v * INV_T)
            a_row[sl] = v * e
            b_row[sl] = e
        pltpu.sync_copy(a_row, num_sp.at[dstt], add=True)
        pltpu.sync_copy(b_row, den_sp.at[dstt], add=True)
        return _
    lax.fori_loop(0, nch - FULL_GROUPS * GE, tail, None)
    plsc.subcore_barrier()

    wb = s * ACC_SL
    pltpu.sync_copy(num_sp.at[pl.ds(wb, ACC_SL)], out_h.at[c, 0, pl.ds(wb, ACC_SL)])
    pltpu.sync_copy(den_sp.at[pl.ds(wb, ACC_SL)], out_h.at[c, 1, pl.ds(wb, ACC_SL)])


_sc_edge_pass = functools.partial(
    pl.kernel,
    out_type=jax.ShapeDtypeStruct((2, 2, ACC), jnp.float32),
    mesh=plsc.VectorSubcoreMesh(core_axis_name="c", subcore_axis_name="s"),
    compiler_params=pltpu.CompilerParams(use_tc_tiling_on_sc=False),
    scratch_types=[
        pltpu.VMEM_SHARED((N_LIT,), jnp.float32),
        pltpu.VMEM_SHARED((ACC,), jnp.float32),
        pltpu.VMEM_SHARED((ACC,), jnp.float32),
        pltpu.VMEM((EMB_CP,), jnp.float32),
        pltpu.VMEM((EMB_CP,), jnp.float32),
        pltpu.VMEM((ZHALF,), jnp.float32),
        pltpu.VMEM((GEL,), jnp.int32),
        pltpu.VMEM((GEL,), jnp.int32),
        pltpu.VMEM((128,), jnp.int32),
        pltpu.VMEM((128,), jnp.int32),
        pltpu.VMEM((GEL,), jnp.float32),
        pltpu.VMEM((GEL,), jnp.float32),
        pltpu.VMEM((GEL,), jnp.float32),
        pltpu.VMEM((128,), jnp.float32),
        pltpu.VMEM((128,), jnp.float32),
        pltpu.VMEM((128,), jnp.float32),
    ],
)(_sc_body)


_FIN_ROWS = ACC // 128


def _fin_body(p_ref, o_ref):
    num = p_ref[0, 0] + p_ref[1, 0]
    den = p_ref[0, 1] + p_ref[1, 1]
    row = lax.broadcasted_iota(jnp.int32, (_FIN_ROWS, 128), 0)
    col = lax.broadcasted_iota(jnp.int32, (_FIN_ROWS, 128), 1)
    idx = row * 128 + col
    valid = idx < N_CLAUSES
    cval = jnp.where(valid & (den > 0), num / jnp.maximum(den, 1e-30), 0.0)
    w2 = jnp.where(valid, jnp.exp(-cval * INV_T), 0.0)
    o_ref[...] = (jnp.sum(cval * w2) / jnp.sum(w2)).reshape(1, 1)


_finalize = pl.pallas_call(
    _fin_body,
    out_shape=jax.ShapeDtypeStruct((1, 1), jnp.float32),
)


def kernel(emb, or_src, or_dst, epoch):
    del epoch
    part = _sc_edge_pass(emb, or_src, or_dst)
    res = _finalize(part.reshape(2, 2, _FIN_ROWS, 128))
    return res[0, 0]


# speedup vs baseline: 728.0753x; 1.3848x over previous
import functools

import jax
import jax.numpy as jnp
from jax import lax
from jax.experimental import pallas as pl
from jax.experimental.pallas import tpu as pltpu
from jax.experimental.pallas import tpu_sc as plsc

NV = 50000
N_LIT = 2 * NV
N_CLAUSES = 200000
E = 1600000
INV_T = float(2.0 ** 1.2)

NCHUNK = E // 128
NW = 32
BASE_CH = NCHUNK // NW
EXTRA = NCHUNK - BASE_CH * NW
GE = 8
GEL = GE * 128
FULL_GROUPS = BASE_CH // GE
KK = FULL_GROUPS // 2

ACC = 200192
ACC_SL = ACC // 16
ZHALF = ACC_SL // 2
EMB_SL = 3120
EMB_CP = 3200


def _sc_body(emb_h, src_h, dst_h, out_h,
             lit_sp, num_sp, den_sp,
             embw, negw, zbuf,
             src_b0, src_b1, dst_b0, dst_b1,
             v_b0, v_b1, a_b0, a_b1, b_b0, b_b1,
             srct, dstt, v_row, a_row, b_row,
             gsem0, gsem1, ssem0, ssem1,
             srcsem0, srcsem1, dstsem0, dstsem1):
    c = lax.axis_index("c")
    s = lax.axis_index("s")
    wid = c * 16 + s

    def z16(i, _):
        zbuf[pl.ds(i * 16, 16)] = jnp.zeros((16,), jnp.float32)
        return _
    lax.fori_loop(0, ZHALF // 16, z16, None)
    zb = s * ACC_SL
    pltpu.sync_copy(zbuf, num_sp.at[pl.ds(zb, ZHALF)])
    pltpu.sync_copy(zbuf, num_sp.at[pl.ds(zb + ZHALF, ZHALF)])
    pltpu.sync_copy(zbuf, den_sp.at[pl.ds(zb, ZHALF)])
    pltpu.sync_copy(zbuf, den_sp.at[pl.ds(zb + ZHALF, ZHALF)])

    off = s * EMB_SL
    pltpu.sync_copy(emb_h.at[pl.ds(off, EMB_CP)], embw)

    def neg(i, _):
        negw[pl.ds(i * 16, 16)] = 1.0 - embw[pl.ds(i * 16, 16)]
        return _
    lax.fori_loop(0, EMB_CP // 16, neg, None)
    pltpu.sync_copy(embw, lit_sp.at[pl.ds(off, EMB_CP)])
    pltpu.sync_copy(negw, lit_sp.at[pl.ds(NV + off, EMB_CP)])
    plsc.subcore_barrier()

    cb = wid * BASE_CH + jnp.minimum(wid, EXTRA)
    nch = jnp.where(wid < EXTRA, BASE_CH + 1, BASE_CH)
    eb = cb * 128

    def src_sl(g):
        return src_h.at[pl.ds(eb + g * GEL, GEL)]

    def dst_sl(g):
        return dst_h.at[pl.ds(eb + g * GEL, GEL)]

    def compute(v_b, a_b, b_b):
        def cstep(i, _):
            sl = pl.ds(i * 16, 16)
            v = v_b[sl]
            e = jnp.exp(v * INV_T)
            a_b[sl] = v * e
            b_b[sl] = e
            return _
        lax.fori_loop(0, GEL // 16, cstep, None)

    pltpu.async_copy(src_sl(0), src_b0, srcsem0)
    pltpu.async_copy(src_sl(1), src_b1, srcsem1)
    pltpu.async_copy(dst_sl(0), dst_b0, dstsem0)
    pltpu.make_async_copy(src_sl(0), src_b0, srcsem0).wait()
    pltpu.async_copy(lit_sp.at[src_b0], v_b0, gsem0)

    def body(k, _):
        g0 = k * 2
        pltpu.make_async_copy(lit_sp.at[src_b0], v_b0, gsem0).wait()

        @pl.when(k < KK - 1)
        def _():
            pltpu.async_copy(src_sl(g0 + 2), src_b0, srcsem0)
        pltpu.make_async_copy(src_sl(g0 + 1), src_b1, srcsem1).wait()
        pltpu.async_copy(lit_sp.at[src_b1], v_b1, gsem1)
        compute(v_b0, a_b0, b_b0)
        pltpu.make_async_copy(dst_sl(g0), dst_b0, dstsem0).wait()

        @pl.when(k > 0)
        def _():
            pltpu.make_async_copy(a_b1, num_sp.at[dst_b1], ssem1).wait()
            pltpu.make_async_copy(b_b1, den_sp.at[dst_b1], ssem1).wait()
        pltpu.async_copy(dst_sl(g0 + 1), dst_b1, dstsem1)
        pltpu.async_copy(a_b0, num_sp.at[dst_b0], ssem0, add=True)
        pltpu.async_copy(b_b0, den_sp.at[dst_b0], ssem0, add=True)

        pltpu.make_async_copy(lit_sp.at[src_b1], v_b1, gsem1).wait()

        @pl.when(k < KK - 1)
        def _():
            pltpu.async_copy(src_sl(g0 + 3), src_b1, srcsem1)
            pltpu.make_async_copy(src_sl(g0 + 2), src_b0, srcsem0).wait()
            pltpu.async_copy(lit_sp.at[src_b0], v_b0, gsem0)
        compute(v_b1, a_b1, b_b1)
        pltpu.make_async_copy(dst_sl(g0 + 1), dst_b1, dstsem1).wait()
        pltpu.make_async_copy(a_b0, num_sp.at[dst_b0], ssem0).wait()
        pltpu.make_async_copy(b_b0, den_sp.at[dst_b0], ssem0).wait()

        @pl.when(k < KK - 1)
        def _():
            pltpu.async_copy(dst_sl(g0 + 2), dst_b0, dstsem0)
        pltpu.async_copy(a_b1, num_sp.at[dst_b1], ssem1, add=True)
        pltpu.async_copy(b_b1, den_sp.at[dst_b1], ssem1, add=True)
        return _
    lax.fori_loop(0, KK, body, None)
    pltpu.make_async_copy(a_b1, num_sp.at[dst_b1], ssem1).wait()
    pltpu.make_async_copy(b_b1, den_sp.at[dst_b1], ssem1).wait()

    def tail(j, _):
        rb = eb + FULL_GROUPS * GEL + j * 128
        pltpu.sync_copy(src_h.at[pl.ds(rb, 128)], srct)
        pltpu.sync_copy(dst_h.at[pl.ds(rb, 128)], dstt)
        pltpu.sync_copy(lit_sp.at[srct], v_row)
        for i in range(8):
            sl = pl.ds(i * 16, 16)
            v = v_row[sl]
            e = jnp.exp(v * INV_T)
            a_row[sl] = v * e
            b_row[sl] = e
        pltpu.sync_copy(a_row, num_sp.at[dstt], add=True)
        pltpu.sync_copy(b_row, den_sp.at[dstt], add=True)
        return _
    lax.fori_loop(0, nch - FULL_GROUPS * GE, tail, None)
    plsc.subcore_barrier()

    wb = s * ACC_SL
    pltpu.sync_copy(num_sp.at[pl.ds(wb, ACC_SL)], out_h.at[c, 0, pl.ds(wb, ACC_SL)])
    pltpu.sync_copy(den_sp.at[pl.ds(wb, ACC_SL)], out_h.at[c, 1, pl.ds(wb, ACC_SL)])


_sc_edge_pass = functools.partial(
    pl.kernel,
    out_type=jax.ShapeDtypeStruct((2, 2, ACC), jnp.float32),
    mesh=plsc.VectorSubcoreMesh(core_axis_name="c", subcore_axis_name="s"),
    compiler_params=pltpu.CompilerParams(use_tc_tiling_on_sc=False),
    scratch_types=[
        pltpu.VMEM_SHARED((N_LIT,), jnp.float32),
        pltpu.VMEM_SHARED((ACC,), jnp.float32),
        pltpu.VMEM_SHARED((ACC,), jnp.float32),
        pltpu.VMEM((EMB_CP,), jnp.float32),
        pltpu.VMEM((EMB_CP,), jnp.float32),
        pltpu.VMEM((ZHALF,), jnp.float32),
        pltpu.VMEM((GEL,), jnp.int32),
        pltpu.VMEM((GEL,), jnp.int32),
        pltpu.VMEM((GEL,), jnp.int32),
        pltpu.VMEM((GEL,), jnp.int32),
        pltpu.VMEM((GEL,), jnp.float32),
        pltpu.VMEM((GEL,), jnp.float32),
        pltpu.VMEM((GEL,), jnp.float32),
        pltpu.VMEM((GEL,), jnp.float32),
        pltpu.VMEM((GEL,), jnp.float32),
        pltpu.VMEM((GEL,), jnp.float32),
        pltpu.VMEM((128,), jnp.int32),
        pltpu.VMEM((128,), jnp.int32),
        pltpu.VMEM((128,), jnp.float32),
        pltpu.VMEM((128,), jnp.float32),
        pltpu.VMEM((128,), jnp.float32),
        pltpu.SemaphoreType.DMA,
        pltpu.SemaphoreType.DMA,
        pltpu.SemaphoreType.DMA,
        pltpu.SemaphoreType.DMA,
        pltpu.SemaphoreType.DMA,
        pltpu.SemaphoreType.DMA,
        pltpu.SemaphoreType.DMA,
        pltpu.SemaphoreType.DMA,
    ],
)(_sc_body)


_FIN_ROWS = ACC // 128


def _fin_body(p_ref, o_ref):
    num = p_ref[0, 0] + p_ref[1, 0]
    den = p_ref[0, 1] + p_ref[1, 1]
    row = lax.broadcasted_iota(jnp.int32, (_FIN_ROWS, 128), 0)
    col = lax.broadcasted_iota(jnp.int32, (_FIN_ROWS, 128), 1)
    idx = row * 128 + col
    valid = idx < N_CLAUSES
    cval = jnp.where(valid & (den > 0), num / jnp.maximum(den, 1e-30), 0.0)
    w2 = jnp.where(valid, jnp.exp(-cval * INV_T), 0.0)
    o_ref[...] = (jnp.sum(cval * w2) / jnp.sum(w2)).reshape(1, 1)


_finalize = pl.pallas_call(
    _fin_body,
    out_shape=jax.ShapeDtypeStruct((1, 1), jnp.float32),
)


def kernel(emb, or_src, or_dst, epoch):
    del epoch
    part = _sc_edge_pass(emb, or_src, or_dst)
    res = _finalize(part.reshape(2, 2, _FIN_ROWS, 128))
    return res[0, 0]


# speedup vs baseline: 754.8404x; 1.0368x over previous
import functools

import jax
import jax.numpy as jnp
from jax import lax
from jax.experimental import pallas as pl
from jax.experimental.pallas import tpu as pltpu
from jax.experimental.pallas import tpu_sc as plsc

NV = 50000
N_LIT = 2 * NV
N_CLAUSES = 200000
E = 1600000
INV_T = float(2.0 ** 1.2)

NCHUNK = E // 128
NW = 32
BASE_CH = NCHUNK // NW
EXTRA = NCHUNK - BASE_CH * NW
GE = 16
GEL = GE * 128
FULL_GROUPS = BASE_CH // GE
KK = FULL_GROUPS // 2

ACC = 200192
ACC_SL = ACC // 16
ZHALF = ACC_SL // 2
EMB_SL = 3120
EMB_CP = 3200


def _sc_body(emb_h, src_h, dst_h, out_h,
             lit_sp, num_sp, den_sp,
             embw, negw, zbuf,
             src_b0, src_b1, dst_b0, dst_b1,
             v_b0, v_b1, a_b0, a_b1, b_b0, b_b1,
             srct, dstt, v_row, a_row, b_row,
             gsem0, gsem1, ssem0, ssem1,
             srcsem0, srcsem1, dstsem0, dstsem1):
    c = lax.axis_index("c")
    s = lax.axis_index("s")
    wid = c * 16 + s

    def z16(i, _):
        zbuf[pl.ds(i * 16, 16)] = jnp.zeros((16,), jnp.float32)
        return _
    lax.fori_loop(0, ZHALF // 16, z16, None)
    zb = s * ACC_SL
    pltpu.sync_copy(zbuf, num_sp.at[pl.ds(zb, ZHALF)])
    pltpu.sync_copy(zbuf, num_sp.at[pl.ds(zb + ZHALF, ZHALF)])
    pltpu.sync_copy(zbuf, den_sp.at[pl.ds(zb, ZHALF)])
    pltpu.sync_copy(zbuf, den_sp.at[pl.ds(zb + ZHALF, ZHALF)])

    off = s * EMB_SL
    pltpu.sync_copy(emb_h.at[pl.ds(off, EMB_CP)], embw)

    def neg(i, _):
        negw[pl.ds(i * 16, 16)] = 1.0 - embw[pl.ds(i * 16, 16)]
        return _
    lax.fori_loop(0, EMB_CP // 16, neg, None)
    pltpu.sync_copy(embw, lit_sp.at[pl.ds(off, EMB_CP)])
    pltpu.sync_copy(negw, lit_sp.at[pl.ds(NV + off, EMB_CP)])
    plsc.subcore_barrier()

    cb = wid * BASE_CH + jnp.minimum(wid, EXTRA)
    nch = jnp.where(wid < EXTRA, BASE_CH + 1, BASE_CH)
    eb = cb * 128

    def src_sl(g):
        return src_h.at[pl.ds(eb + g * GEL, GEL)]

    def dst_sl(g):
        return dst_h.at[pl.ds(eb + g * GEL, GEL)]

    def compute(v_b, a_b, b_b):
        def cstep(i, _):
            sl = pl.ds(i * 16, 16)
            v = v_b[sl]
            e = jnp.exp(v * INV_T)
            a_b[sl] = v * e
            b_b[sl] = e
            return _
        lax.fori_loop(0, GEL // 16, cstep, None)

    pltpu.async_copy(src_sl(0), src_b0, srcsem0)
    pltpu.async_copy(src_sl(1), src_b1, srcsem1)
    pltpu.async_copy(dst_sl(0), dst_b0, dstsem0)
    pltpu.make_async_copy(src_sl(0), src_b0, srcsem0).wait()
    pltpu.async_copy(lit_sp.at[src_b0], v_b0, gsem0)

    def body(k, _):
        g0 = k * 2
        pltpu.make_async_copy(lit_sp.at[src_b0], v_b0, gsem0).wait()

        @pl.when(k < KK - 1)
        def _():
            pltpu.async_copy(src_sl(g0 + 2), src_b0, srcsem0)
        pltpu.make_async_copy(src_sl(g0 + 1), src_b1, srcsem1).wait()
        pltpu.async_copy(lit_sp.at[src_b1], v_b1, gsem1)
        compute(v_b0, a_b0, b_b0)
        pltpu.make_async_copy(dst_sl(g0), dst_b0, dstsem0).wait()

        @pl.when(k > 0)
        def _():
            pltpu.make_async_copy(a_b1, num_sp.at[dst_b1], ssem1).wait()
            pltpu.make_async_copy(b_b1, den_sp.at[dst_b1], ssem1).wait()
        pltpu.async_copy(dst_sl(g0 + 1), dst_b1, dstsem1)
        pltpu.async_copy(a_b0, num_sp.at[dst_b0], ssem0, add=True)
        pltpu.async_copy(b_b0, den_sp.at[dst_b0], ssem0, add=True)

        pltpu.make_async_copy(lit_sp.at[src_b1], v_b1, gsem1).wait()

        @pl.when(k < KK - 1)
        def _():
            pltpu.async_copy(src_sl(g0 + 3), src_b1, srcsem1)
            pltpu.make_async_copy(src_sl(g0 + 2), src_b0, srcsem0).wait()
            pltpu.async_copy(lit_sp.at[src_b0], v_b0, gsem0)
        compute(v_b1, a_b1, b_b1)
        pltpu.make_async_copy(dst_sl(g0 + 1), dst_b1, dstsem1).wait()
        pltpu.make_async_copy(a_b0, num_sp.at[dst_b0], ssem0).wait()
        pltpu.make_async_copy(b_b0, den_sp.at[dst_b0], ssem0).wait()

        @pl.when(k < KK - 1)
        def _():
            pltpu.async_copy(dst_sl(g0 + 2), dst_b0, dstsem0)
        pltpu.async_copy(a_b1, num_sp.at[dst_b1], ssem1, add=True)
        pltpu.async_copy(b_b1, den_sp.at[dst_b1], ssem1, add=True)
        return _
    lax.fori_loop(0, KK, body, None)
    pltpu.make_async_copy(a_b1, num_sp.at[dst_b1], ssem1).wait()
    pltpu.make_async_copy(b_b1, den_sp.at[dst_b1], ssem1).wait()

    def tail(j, _):
        rb = eb + FULL_GROUPS * GEL + j * 128
        pltpu.sync_copy(src_h.at[pl.ds(rb, 128)], srct)
        pltpu.sync_copy(dst_h.at[pl.ds(rb, 128)], dstt)
        pltpu.sync_copy(lit_sp.at[srct], v_row)
        for i in range(8):
            sl = pl.ds(i * 16, 16)
            v = v_row[sl]
            e = jnp.exp(v * INV_T)
            a_row[sl] = v * e
            b_row[sl] = e
        pltpu.sync_copy(a_row, num_sp.at[dstt], add=True)
        pltpu.sync_copy(b_row, den_sp.at[dstt], add=True)
        return _
    lax.fori_loop(0, nch - FULL_GROUPS * GE, tail, None)
    plsc.subcore_barrier()

    wb = s * ACC_SL
    pltpu.sync_copy(num_sp.at[pl.ds(wb, ACC_SL)], out_h.at[c, 0, pl.ds(wb, ACC_SL)])
    pltpu.sync_copy(den_sp.at[pl.ds(wb, ACC_SL)], out_h.at[c, 1, pl.ds(wb, ACC_SL)])


_sc_edge_pass = functools.partial(
    pl.kernel,
    out_type=jax.ShapeDtypeStruct((2, 2, ACC), jnp.float32),
    mesh=plsc.VectorSubcoreMesh(core_axis_name="c", subcore_axis_name="s"),
    compiler_params=pltpu.CompilerParams(use_tc_tiling_on_sc=False),
    scratch_types=[
        pltpu.VMEM_SHARED((N_LIT,), jnp.float32),
        pltpu.VMEM_SHARED((ACC,), jnp.float32),
        pltpu.VMEM_SHARED((ACC,), jnp.float32),
        pltpu.VMEM((EMB_CP,), jnp.float32),
        pltpu.VMEM((EMB_CP,), jnp.float32),
        pltpu.VMEM((ZHALF,), jnp.float32),
        pltpu.VMEM((GEL,), jnp.int32),
        pltpu.VMEM((GEL,), jnp.int32),
        pltpu.VMEM((GEL,), jnp.int32),
        pltpu.VMEM((GEL,), jnp.int32),
        pltpu.VMEM((GEL,), jnp.float32),
        pltpu.VMEM((GEL,), jnp.float32),
        pltpu.VMEM((GEL,), jnp.float32),
        pltpu.VMEM((GEL,), jnp.float32),
        pltpu.VMEM((GEL,), jnp.float32),
        pltpu.VMEM((GEL,), jnp.float32),
        pltpu.VMEM((128,), jnp.int32),
        pltpu.VMEM((128,), jnp.int32),
        pltpu.VMEM((128,), jnp.float32),
        pltpu.VMEM((128,), jnp.float32),
        pltpu.VMEM((128,), jnp.float32),
        pltpu.SemaphoreType.DMA,
        pltpu.SemaphoreType.DMA,
        pltpu.SemaphoreType.DMA,
        pltpu.SemaphoreType.DMA,
        pltpu.SemaphoreType.DMA,
        pltpu.SemaphoreType.DMA,
        pltpu.SemaphoreType.DMA,
        pltpu.SemaphoreType.DMA,
    ],
)(_sc_body)


_FIN_ROWS = ACC // 128


def _fin_body(p_ref, o_ref):
    num = p_ref[0, 0] + p_ref[1, 0]
    den = p_ref[0, 1] + p_ref[1, 1]
    row = lax.broadcasted_iota(jnp.int32, (_FIN_ROWS, 128), 0)
    col = lax.broadcasted_iota(jnp.int32, (_FIN_ROWS, 128), 1)
    idx = row * 128 + col
    valid = idx < N_CLAUSES
    cval = jnp.where(valid & (den > 0), num / jnp.maximum(den, 1e-30), 0.0)
    w2 = jnp.where(valid, jnp.exp(-cval * INV_T), 0.0)
    o_ref[...] = (jnp.sum(cval * w2) / jnp.sum(w2)).reshape(1, 1)


_finalize = pl.pallas_call(
    _fin_body,
    out_shape=jax.ShapeDtypeStruct((1, 1), jnp.float32),
)


def kernel(emb, or_src, or_dst, epoch):
    del epoch
    part = _sc_edge_pass(emb, or_src, or_dst)
    res = _finalize(part.reshape(2, 2, _FIN_ROWS, 128))
    return res[0, 0]


# speedup vs baseline: 841.1700x; 1.1144x over previous
import functools

import jax
import jax.numpy as jnp
from jax import lax
from jax.experimental import pallas as pl
from jax.experimental.pallas import tpu as pltpu
from jax.experimental.pallas import tpu_sc as plsc

NV = 50000
N_LIT = 2 * NV
N_CLAUSES = 200000
E = 1600000
INV_T = float(2.0 ** 1.2)

NCHUNK = E // 128
NW = 32
BASE_CH = NCHUNK // NW
EXTRA = NCHUNK - BASE_CH * NW
GE = 32
GEL = GE * 128
FULL_GROUPS = BASE_CH // GE
KK = FULL_GROUPS // 2

ACC = 200192
ACC_SL = ACC // 16
ZHALF = ACC_SL // 2
EMB_SL = 3120
EMB_CP = 3200


def _sc_body(emb_h, src_h, dst_h, out_h,
             lit_sp, num_sp, den_sp,
             embw, negw, zbuf,
             src_b0, src_b1, dst_b0, dst_b1,
             v_b0, v_b1, a_b0, a_b1, b_b0, b_b1,
             srct, dstt, v_row, a_row, b_row,
             gsem0, gsem1, ssem0, ssem1,
             srcsem0, srcsem1, dstsem0, dstsem1, zsem, esem):
    c = lax.axis_index("c")
    s = lax.axis_index("s")
    wid = c * 16 + s

    cb = wid * BASE_CH + jnp.minimum(wid, EXTRA)
    nch = jnp.where(wid < EXTRA, BASE_CH + 1, BASE_CH)
    eb = cb * 128

    def src_sl(g):
        return src_h.at[pl.ds(eb + g * GEL, GEL)]

    def dst_sl(g):
        return dst_h.at[pl.ds(eb + g * GEL, GEL)]

    pltpu.async_copy(src_sl(0), src_b0, srcsem0)
    pltpu.async_copy(src_sl(1), src_b1, srcsem1)
    pltpu.async_copy(dst_sl(0), dst_b0, dstsem0)
    pltpu.async_copy(emb_h.at[pl.ds(s * EMB_SL, EMB_CP)], embw, esem)

    def z16(i, _):
        zbuf[pl.ds(i * 16, 16)] = jnp.zeros((16,), jnp.float32)
        return _
    lax.fori_loop(0, ZHALF // 16, z16, None)
    zb = s * ACC_SL
    pltpu.async_copy(zbuf, num_sp.at[pl.ds(zb, ZHALF)], zsem)
    pltpu.async_copy(zbuf, num_sp.at[pl.ds(zb + ZHALF, ZHALF)], zsem)
    pltpu.async_copy(zbuf, den_sp.at[pl.ds(zb, ZHALF)], zsem)
    pltpu.async_copy(zbuf, den_sp.at[pl.ds(zb + ZHALF, ZHALF)], zsem)

    off = s * EMB_SL
    pltpu.make_async_copy(emb_h.at[pl.ds(off, EMB_CP)], embw, esem).wait()

    def neg(i, _):
        negw[pl.ds(i * 16, 16)] = 1.0 - embw[pl.ds(i * 16, 16)]
        return _
    lax.fori_loop(0, EMB_CP // 16, neg, None)
    pltpu.async_copy(embw, lit_sp.at[pl.ds(off, EMB_CP)], zsem)
    pltpu.async_copy(negw, lit_sp.at[pl.ds(NV + off, EMB_CP)], zsem)
    for _ in range(4):
        pltpu.make_async_copy(zbuf, num_sp.at[pl.ds(zb, ZHALF)], zsem).wait()
    pltpu.make_async_copy(embw, lit_sp.at[pl.ds(off, EMB_CP)], zsem).wait()
    pltpu.make_async_copy(negw, lit_sp.at[pl.ds(NV + off, EMB_CP)], zsem).wait()
    plsc.subcore_barrier()

    def compute(v_b, a_b, b_b):
        def cstep(i, _):
            sl = pl.ds(i * 16, 16)
            v = v_b[sl]
            e = jnp.exp(v * INV_T)
            a_b[sl] = v * e
            b_b[sl] = e
            return _
        lax.fori_loop(0, GEL // 16, cstep, None)

    pltpu.make_async_copy(src_sl(0), src_b0, srcsem0).wait()
    pltpu.async_copy(lit_sp.at[src_b0], v_b0, gsem0)

    def body(k, _):
        g0 = k * 2
        pltpu.make_async_copy(lit_sp.at[src_b0], v_b0, gsem0).wait()

        @pl.when(k < KK - 1)
        def _():
            pltpu.async_copy(src_sl(g0 + 2), src_b0, srcsem0)
        pltpu.make_async_copy(src_sl(g0 + 1), src_b1, srcsem1).wait()
        pltpu.async_copy(lit_sp.at[src_b1], v_b1, gsem1)
        compute(v_b0, a_b0, b_b0)
        pltpu.make_async_copy(dst_sl(g0), dst_b0, dstsem0).wait()

        @pl.when(k > 0)
        def _():
            pltpu.make_async_copy(a_b1, num_sp.at[dst_b1], ssem1).wait()
            pltpu.make_async_copy(b_b1, den_sp.at[dst_b1], ssem1).wait()
        pltpu.async_copy(dst_sl(g0 + 1), dst_b1, dstsem1)
        pltpu.async_copy(a_b0, num_sp.at[dst_b0], ssem0, add=True)
        pltpu.async_copy(b_b0, den_sp.at[dst_b0], ssem0, add=True)

        pltpu.make_async_copy(lit_sp.at[src_b1], v_b1, gsem1).wait()

        @pl.when(k < KK - 1)
        def _():
            pltpu.async_copy(src_sl(g0 + 3), src_b1, srcsem1)
            pltpu.make_async_copy(src_sl(g0 + 2), src_b0, srcsem0).wait()
            pltpu.async_copy(lit_sp.at[src_b0], v_b0, gsem0)
        compute(v_b1, a_b1, b_b1)
        pltpu.make_async_copy(dst_sl(g0 + 1), dst_b1, dstsem1).wait()
        pltpu.make_async_copy(a_b0, num_sp.at[dst_b0], ssem0).wait()
        pltpu.make_async_copy(b_b0, den_sp.at[dst_b0], ssem0).wait()

        @pl.when(k < KK - 1)
        def _():
            pltpu.async_copy(dst_sl(g0 + 2), dst_b0, dstsem0)
        pltpu.async_copy(a_b1, num_sp.at[dst_b1], ssem1, add=True)
        pltpu.async_copy(b_b1, den_sp.at[dst_b1], ssem1, add=True)
        return _
    lax.fori_loop(0, KK, body, None)
    pltpu.make_async_copy(a_b1, num_sp.at[dst_b1], ssem1).wait()
    pltpu.make_async_copy(b_b1, den_sp.at[dst_b1], ssem1).wait()

    def tail(j, _):
        rb = eb + FULL_GROUPS * GEL + j * 128
        pltpu.sync_copy(src_h.at[pl.ds(rb, 128)], srct)
        pltpu.sync_copy(dst_h.at[pl.ds(rb, 128)], dstt)
        pltpu.sync_copy(lit_sp.at[srct], v_row)
        for i in range(8):
            sl = pl.ds(i * 16, 16)
            v = v_row[sl]
            e = jnp.exp(v * INV_T)
            a_row[sl] = v * e
            b_row[sl] = e
        pltpu.sync_copy(a_row, num_sp.at[dstt], add=True)
        pltpu.sync_copy(b_row, den_sp.at[dstt], add=True)
        return _
    lax.fori_loop(0, nch - FULL_GROUPS * GE, tail, None)
    plsc.subcore_barrier()

    wb = s * ACC_SL
    pltpu.sync_copy(num_sp.at[pl.ds(wb, ACC_SL)], out_h.at[c, 0, pl.ds(wb, ACC_SL)])
    pltpu.sync_copy(den_sp.at[pl.ds(wb, ACC_SL)], out_h.at[c, 1, pl.ds(wb, ACC_SL)])


_sc_edge_pass = functools.partial(
    pl.kernel,
    out_type=jax.ShapeDtypeStruct((2, 2, ACC), jnp.float32),
    mesh=plsc.VectorSubcoreMesh(core_axis_name="c", subcore_axis_name="s"),
    compiler_params=pltpu.CompilerParams(use_tc_tiling_on_sc=False),
    scratch_types=[
        pltpu.VMEM_SHARED((N_LIT,), jnp.float32),
        pltpu.VMEM_SHARED((ACC,), jnp.float32),
        pltpu.VMEM_SHARED((ACC,), jnp.float32),
        pltpu.VMEM((EMB_CP,), jnp.float32),
        pltpu.VMEM((EMB_CP,), jnp.float32),
        pltpu.VMEM((ZHALF,), jnp.float32),
        pltpu.VMEM((GEL,), jnp.int32),
        pltpu.VMEM((GEL,), jnp.int32),
        pltpu.VMEM((GEL,), jnp.int32),
        pltpu.VMEM((GEL,), jnp.int32),
        pltpu.VMEM((GEL,), jnp.float32),
        pltpu.VMEM((GEL,), jnp.float32),
        pltpu.VMEM((GEL,), jnp.float32),
        pltpu.VMEM((GEL,), jnp.float32),
        pltpu.VMEM((GEL,), jnp.float32),
        pltpu.VMEM((GEL,), jnp.float32),
        pltpu.VMEM((128,), jnp.int32),
        pltpu.VMEM((128,), jnp.int32),
        pltpu.VMEM((128,), jnp.float32),
        pltpu.VMEM((128,), jnp.float32),
        pltpu.VMEM((128,), jnp.float32),
        pltpu.SemaphoreType.DMA,
        pltpu.SemaphoreType.DMA,
        pltpu.SemaphoreType.DMA,
        pltpu.SemaphoreType.DMA,
        pltpu.SemaphoreType.DMA,
        pltpu.SemaphoreType.DMA,
        pltpu.SemaphoreType.DMA,
        pltpu.SemaphoreType.DMA,
        pltpu.SemaphoreType.DMA,
        pltpu.SemaphoreType.DMA,
    ],
)(_sc_body)


_FIN_ROWS = ACC // 128


def _fin_body(p_ref, o_ref):
    num = p_ref[0, 0] + p_ref[1, 0]
    den = p_ref[0, 1] + p_ref[1, 1]
    row = lax.broadcasted_iota(jnp.int32, (_FIN_ROWS, 128), 0)
    col = lax.broadcasted_iota(jnp.int32, (_FIN_ROWS, 128), 1)
    idx = row * 128 + col
    valid = idx < N_CLAUSES
    cval = jnp.where(valid & (den > 0), num / jnp.maximum(den, 1e-30), 0.0)
    w2 = jnp.where(valid, jnp.exp(-cval * INV_T), 0.0)
    o_ref[...] = (jnp.sum(cval * w2) / jnp.sum(w2)).reshape(1, 1)


_finalize = pl.pallas_call(
    _fin_body,
    out_shape=jax.ShapeDtypeStruct((1, 1), jnp.float32),
)


def kernel(emb, or_src, or_dst, epoch):
    del epoch
    part = _sc_edge_pass(emb, or_src, or_dst)
    res = _finalize(part.reshape(2, 2, _FIN_ROWS, 128))
    return res[0, 0]


# speedup vs baseline: 923.8979x; 1.0983x over previous
import functools

import jax
import jax.numpy as jnp
from jax import lax
from jax.experimental import pallas as pl
from jax.experimental.pallas import tpu as pltpu
from jax.experimental.pallas import tpu_sc as plsc

NV = 50000
N_LIT = 2 * NV
N_CLAUSES = 200000
E = 1600000
INV_T = float(2.0 ** 1.2)

NCHUNK = E // 128
NW = 32
BASE_CH = NCHUNK // NW
EXTRA = NCHUNK - BASE_CH * NW
GE = 64
GEL = GE * 128
FULL_GROUPS = BASE_CH // GE
KK = FULL_GROUPS // 2

ACC = 200192
ACC_SL = ACC // 16
ZHALF = ACC_SL // 2
EMB_SL = 3120
EMB_CP = 3200


def _sc_body(emb_h, src_h, dst_h, out_h,
             lit_sp, num_sp, den_sp,
             embw, negw, zbuf,
             src_b0, src_b1, dst_b0, dst_b1,
             v_b0, v_b1, a_b0, a_b1, b_b0, b_b1,
             srct, dstt, v_row, a_row, b_row,
             gsem0, gsem1, ssem0, ssem1,
             srcsem0, srcsem1, dstsem0, dstsem1, zsem, esem):
    c = lax.axis_index("c")
    s = lax.axis_index("s")
    wid = c * 16 + s

    cb = wid * BASE_CH + jnp.minimum(wid, EXTRA)
    nch = jnp.where(wid < EXTRA, BASE_CH + 1, BASE_CH)
    eb = cb * 128

    def src_sl(g):
        return src_h.at[pl.ds(eb + g * GEL, GEL)]

    def dst_sl(g):
        return dst_h.at[pl.ds(eb + g * GEL, GEL)]

    pltpu.async_copy(src_sl(0), src_b0, srcsem0)
    pltpu.async_copy(src_sl(1), src_b1, srcsem1)
    pltpu.async_copy(dst_sl(0), dst_b0, dstsem0)
    pltpu.async_copy(emb_h.at[pl.ds(s * EMB_SL, EMB_CP)], embw, esem)

    def z16(i, _):
        zbuf[pl.ds(i * 16, 16)] = jnp.zeros((16,), jnp.float32)
        return _
    lax.fori_loop(0, ZHALF // 16, z16, None)
    zb = s * ACC_SL
    pltpu.async_copy(zbuf, num_sp.at[pl.ds(zb, ZHALF)], zsem)
    pltpu.async_copy(zbuf, num_sp.at[pl.ds(zb + ZHALF, ZHALF)], zsem)
    pltpu.async_copy(zbuf, den_sp.at[pl.ds(zb, ZHALF)], zsem)
    pltpu.async_copy(zbuf, den_sp.at[pl.ds(zb + ZHALF, ZHALF)], zsem)

    off = s * EMB_SL
    pltpu.make_async_copy(emb_h.at[pl.ds(off, EMB_CP)], embw, esem).wait()

    def neg(i, _):
        negw[pl.ds(i * 16, 16)] = 1.0 - embw[pl.ds(i * 16, 16)]
        return _
    lax.fori_loop(0, EMB_CP // 16, neg, None)
    pltpu.async_copy(embw, lit_sp.at[pl.ds(off, EMB_CP)], zsem)
    pltpu.async_copy(negw, lit_sp.at[pl.ds(NV + off, EMB_CP)], zsem)
    for _ in range(4):
        pltpu.make_async_copy(zbuf, num_sp.at[pl.ds(zb, ZHALF)], zsem).wait()
    pltpu.make_async_copy(embw, lit_sp.at[pl.ds(off, EMB_CP)], zsem).wait()
    pltpu.make_async_copy(negw, lit_sp.at[pl.ds(NV + off, EMB_CP)], zsem).wait()
    plsc.subcore_barrier()

    def compute(v_b, a_b, b_b):
        def cstep(i, _):
            for u in range(4):
                sl = pl.ds(i * 64 + u * 16, 16)
                v = v_b[sl]
                e = jnp.exp(v * INV_T)
                a_b[sl] = v * e
                b_b[sl] = e
            return _
        lax.fori_loop(0, GEL // 64, cstep, None)

    pltpu.make_async_copy(src_sl(0), src_b0, srcsem0).wait()
    pltpu.async_copy(lit_sp.at[src_b0], v_b0, gsem0)

    def body(k, _):
        g0 = k * 2
        pltpu.make_async_copy(lit_sp.at[src_b0], v_b0, gsem0).wait()

        @pl.when(k < KK - 1)
        def _():
            pltpu.async_copy(src_sl(g0 + 2), src_b0, srcsem0)
        pltpu.make_async_copy(src_sl(g0 + 1), src_b1, srcsem1).wait()
        pltpu.async_copy(lit_sp.at[src_b1], v_b1, gsem1)
        compute(v_b0, a_b0, b_b0)
        pltpu.make_async_copy(dst_sl(g0), dst_b0, dstsem0).wait()

        @pl.when(k > 0)
        def _():
            pltpu.make_async_copy(a_b1, num_sp.at[dst_b1], ssem1).wait()
            pltpu.make_async_copy(b_b1, den_sp.at[dst_b1], ssem1).wait()
        pltpu.async_copy(dst_sl(g0 + 1), dst_b1, dstsem1)
        pltpu.async_copy(a_b0, num_sp.at[dst_b0], ssem0, add=True)
        pltpu.async_copy(b_b0, den_sp.at[dst_b0], ssem0, add=True)

        pltpu.make_async_copy(lit_sp.at[src_b1], v_b1, gsem1).wait()

        @pl.when(k < KK - 1)
        def _():
            pltpu.async_copy(src_sl(g0 + 3), src_b1, srcsem1)
            pltpu.make_async_copy(src_sl(g0 + 2), src_b0, srcsem0).wait()
            pltpu.async_copy(lit_sp.at[src_b0], v_b0, gsem0)
        compute(v_b1, a_b1, b_b1)
        pltpu.make_async_copy(dst_sl(g0 + 1), dst_b1, dstsem1).wait()
        pltpu.make_async_copy(a_b0, num_sp.at[dst_b0], ssem0).wait()
        pltpu.make_async_copy(b_b0, den_sp.at[dst_b0], ssem0).wait()

        @pl.when(k < KK - 1)
        def _():
            pltpu.async_copy(dst_sl(g0 + 2), dst_b0, dstsem0)
        pltpu.async_copy(a_b1, num_sp.at[dst_b1], ssem1, add=True)
        pltpu.async_copy(b_b1, den_sp.at[dst_b1], ssem1, add=True)
        return _
    lax.fori_loop(0, KK, body, None)
    pltpu.make_async_copy(a_b1, num_sp.at[dst_b1], ssem1).wait()
    pltpu.make_async_copy(b_b1, den_sp.at[dst_b1], ssem1).wait()

    def tail(j, _):
        rb = eb + FULL_GROUPS * GEL + j * 128
        pltpu.sync_copy(src_h.at[pl.ds(rb, 128)], srct)
        pltpu.sync_copy(dst_h.at[pl.ds(rb, 128)], dstt)
        pltpu.sync_copy(lit_sp.at[srct], v_row)
        for i in range(8):
            sl = pl.ds(i * 16, 16)
            v = v_row[sl]
            e = jnp.exp(v * INV_T)
            a_row[sl] = v * e
            b_row[sl] = e
        pltpu.sync_copy(a_row, num_sp.at[dstt], add=True)
        pltpu.sync_copy(b_row, den_sp.at[dstt], add=True)
        return _
    lax.fori_loop(0, nch - FULL_GROUPS * GE, tail, None)
    plsc.subcore_barrier()

    wb = s * ACC_SL
    pltpu.sync_copy(num_sp.at[pl.ds(wb, ACC_SL)], out_h.at[c, 0, pl.ds(wb, ACC_SL)])
    pltpu.sync_copy(den_sp.at[pl.ds(wb, ACC_SL)], out_h.at[c, 1, pl.ds(wb, ACC_SL)])


_sc_edge_pass = functools.partial(
    pl.kernel,
    out_type=jax.ShapeDtypeStruct((2, 2, ACC), jnp.float32),
    mesh=plsc.VectorSubcoreMesh(core_axis_name="c", subcore_axis_name="s"),
    compiler_params=pltpu.CompilerParams(use_tc_tiling_on_sc=False),
    scratch_types=[
        pltpu.VMEM_SHARED((N_LIT,), jnp.float32),
        pltpu.VMEM_SHARED((ACC,), jnp.float32),
        pltpu.VMEM_SHARED((ACC,), jnp.float32),
        pltpu.VMEM((EMB_CP,), jnp.float32),
        pltpu.VMEM((EMB_CP,), jnp.float32),
        pltpu.VMEM((ZHALF,), jnp.float32),
        pltpu.VMEM((GEL,), jnp.int32),
        pltpu.VMEM((GEL,), jnp.int32),
        pltpu.VMEM((GEL,), jnp.int32),
        pltpu.VMEM((GEL,), jnp.int32),
        pltpu.VMEM((GEL,), jnp.float32),
        pltpu.VMEM((GEL,), jnp.float32),
        pltpu.VMEM((GEL,), jnp.float32),
        pltpu.VMEM((GEL,), jnp.float32),
        pltpu.VMEM((GEL,), jnp.float32),
        pltpu.VMEM((GEL,), jnp.float32),
        pltpu.VMEM((128,), jnp.int32),
        pltpu.VMEM((128,), jnp.int32),
        pltpu.VMEM((128,), jnp.float32),
        pltpu.VMEM((128,), jnp.float32),
        pltpu.VMEM((128,), jnp.float32),
        pltpu.SemaphoreType.DMA,
        pltpu.SemaphoreType.DMA,
        pltpu.SemaphoreType.DMA,
        pltpu.SemaphoreType.DMA,
        pltpu.SemaphoreType.DMA,
        pltpu.SemaphoreType.DMA,
        pltpu.SemaphoreType.DMA,
        pltpu.SemaphoreType.DMA,
        pltpu.SemaphoreType.DMA,
        pltpu.SemaphoreType.DMA,
    ],
)(_sc_body)


_FIN_ROWS = ACC // 128


def _fin_body(p_ref, o_ref):
    num = p_ref[0, 0] + p_ref[1, 0]
    den = p_ref[0, 1] + p_ref[1, 1]
    row = lax.broadcasted_iota(jnp.int32, (_FIN_ROWS, 128), 0)
    col = lax.broadcasted_iota(jnp.int32, (_FIN_ROWS, 128), 1)
    idx = row * 128 + col
    valid = idx < N_CLAUSES
    cval = jnp.where(valid & (den > 0), num / jnp.maximum(den, 1e-30), 0.0)
    w2 = jnp.where(valid, jnp.exp(-cval * INV_T), 0.0)
    o_ref[...] = (jnp.sum(cval * w2) / jnp.sum(w2)).reshape(1, 1)


_finalize = pl.pallas_call(
    _fin_body,
    out_shape=jax.ShapeDtypeStruct((1, 1), jnp.float32),
)


def kernel(emb, or_src, or_dst, epoch):
    del epoch
    part = _sc_edge_pass(emb, or_src, or_dst)
    res = _finalize(part.reshape(2, 2, _FIN_ROWS, 128))
    return res[0, 0]
